# baseline (speedup 1.0000x reference)
import jax
import jax.numpy as jnp
import numpy as np
from jax.experimental import pallas as pl

EPS = 1e-05


def _bn(x, g, b):
    return g * x / jnp.sqrt(1.0 + EPS) + b


def _knn_idx(q, p, k):
    d = jnp.sum(q * q, -1)[:, None] + jnp.sum(p * p, -1)[None, :] - 2.0 * (q @ p.T)
    _, idx = jax.lax.top_k(-d, k)
    return idx


def _fps(p, m):
    def body(i, state):
        idxs, dist = state
        d = jnp.sum((p - p[idxs[i - 1]]) ** 2, axis=1)
        dist = jnp.minimum(dist, d)
        idxs = idxs.at[i].set(jnp.argmax(dist).astype(jnp.int32))
        return (idxs, dist)
    idxs = jnp.zeros((m,), jnp.int32)
    dist = jnp.sum((p - p[0]) ** 2, axis=1)
    idxs, _ = jax.lax.fori_loop(1, m, body, (idxs, dist))
    return idxs


def _pt_layer(p, x, P, ns, s):
    xq = x @ P['Wq'] + P['bq']
    xk = x @ P['Wk'] + P['bk']
    xv = x @ P['Wv'] + P['bv']
    idx = _knn_idx(p, p, ns)
    pr = p[idx] - p[:, None, :]
    xkg = xk[idx]
    xvg = xv[idx]
    pe = pr @ P['Wp1'] + P['bp1']
    pe = jax.nn.relu(_bn(pe, P['gp'], P['bp']))
    pe = pe @ P['Wp2'] + P['bp2']
    w = xkg - xq[:, None, :] + pe
    w = jax.nn.relu(_bn(w, P['gw1'], P['bw1']))
    w = w @ P['Ww1'] + P['cw1']
    w = jax.nn.relu(_bn(w, P['gw2'], P['bw2']))
    w = w @ P['Ww2'] + P['cw2']
    w = jax.nn.softmax(w, axis=1)
    n, nsp, c = xvg.shape
    return ((xvg + pe).reshape(n, nsp, s, c // s) * w[:, :, None, :]).sum(1).reshape(n, c)


def _block(p, x, P, ns, s):
    h = jax.nn.relu(_bn(x @ P['W1'], P['g1'], P['b1']))
    h = jax.nn.relu(_bn(_pt_layer(p, h, P['tr'], ns, s), P['g2'], P['b2']))
    h = _bn(h @ P['W3'], P['g3'], P['b3'])
    return jax.nn.relu(h + x)


def _td_down(p, x, P, ns, idx):
    npnt = p[idx]
    nn = _knn_idx(npnt, p, ns)
    g = jnp.concatenate([p[nn] - npnt[:, None, :], x[nn]], axis=-1)
    h = jax.nn.relu(_bn(g @ P['W'], P['g'], P['b']))
    return npnt, jnp.max(h, axis=1)


def _point_embed(p):
    e = (2.0 ** np.arange(8)).astype(np.float32) * np.pi
    basis = np.zeros((3, 24), np.float32)
    basis[0, :8] = e
    basis[1, 8:16] = e
    basis[2, 16:] = e
    proj = p @ jnp.asarray(basis)
    return jnp.concatenate([jnp.sin(proj), jnp.cos(proj)], axis=-1)


def _relu_pallas(x):
    def body(x_ref, o_ref):
        o_ref[...] = jnp.maximum(x_ref[...], 0.0)
    return pl.pallas_call(
        body, out_shape=jax.ShapeDtypeStruct(x.shape, x.dtype))(x)


def kernel(x, params):
    p0 = x[0]
    idx1 = _fps(p0, p0.shape[0] // 4)
    idx2 = _fps(p0[idx1], p0.shape[0] // 16)
    x0 = jnp.concatenate([p0, _point_embed(p0)], axis=-1)
    h = jax.nn.relu(_bn(x0 @ params['td0']['W'], params['td0']['g'], params['td0']['b']))
    x1 = _block(p0, h, params['b0_0'], 8, 8)
    p2, h = _td_down(p0, x1, params['td1'], 16, idx1)
    h = _block(p2, h, params['b1_0'], 16, 8)
    x2 = _block(p2, h, params['b1_1'], 16, 8)
    p3, h = _td_down(p2, x2, params['td2'], 16, idx2)
    h = _block(p3, h, params['b2_0'], 16, 8)
    h = _block(p3, h, params['b2_1'], 16, 8)
    x3 = _relu_pallas(_block(p3, h, params['b2_2'], 16, 8))
    return (p0, x1, p2, x2, p3, x3)


# speedup vs baseline: 2.4580x; 2.4580x over previous
import functools

import jax
import jax.numpy as jnp
import numpy as np
from jax import lax
from jax.experimental import pallas as pl

EPS = 1e-05


def _fps_kernel_body(m, n8, p_ref, out_ref):
    px = p_ref[0]
    py = p_ref[1]
    pz = p_ref[2]
    m8 = m // 8
    fi = (lax.broadcasted_iota(jnp.int32, (8, n8), 0) * n8
          + lax.broadcasted_iota(jnp.int32, (8, n8), 1))
    fm = (lax.broadcasted_iota(jnp.int32, (8, m8), 0) * m8
          + lax.broadcasted_iota(jnp.int32, (8, m8), 1))

    def step(i, state):
        dist, qx, qy, qz, idxs = state
        dx = px - qx
        dy = py - qy
        dz = pz - qz
        d = (dx * dx + dy * dy) + dz * dz
        dist = jnp.minimum(dist, d)
        mx = jnp.max(dist)
        idx = jnp.min(jnp.where(dist == mx, fi, jnp.int32(8 * n8)))
        sel = (fi == idx).astype(jnp.float32)
        qx = jnp.sum(px * sel)
        qy = jnp.sum(py * sel)
        qz = jnp.sum(pz * sel)
        idxs = jnp.where(fm == i, idx, idxs)
        return (dist, qx, qy, qz, idxs)

    dist0 = jnp.full((8, n8), jnp.inf, jnp.float32)
    idxs0 = jnp.zeros((8, m8), jnp.int32)
    state = (dist0, px[0, 0], py[0, 0], pz[0, 0], idxs0)
    state = lax.fori_loop(1, m, step, state)
    out_ref[...] = state[4]


def _fps(p, m):
    n = p.shape[0]
    pxyz = p.T.reshape(3, 8, n // 8)
    out = pl.pallas_call(
        functools.partial(_fps_kernel_body, m, n // 8),
        out_shape=jax.ShapeDtypeStruct((8, m // 8), jnp.int32),
    )(pxyz)
    return out.reshape(m)


def _bn(x, g, b):
    return g * x / jnp.sqrt(1.0 + EPS) + b


def _knn_idx(q, p, k):
    d = jnp.sum(q * q, -1)[:, None] + jnp.sum(p * p, -1)[None, :] - 2.0 * (q @ p.T)
    _, idx = jax.lax.top_k(-d, k)
    return idx


def _pt_layer(p, x, P, ns, s):
    xq = x @ P['Wq'] + P['bq']
    xk = x @ P['Wk'] + P['bk']
    xv = x @ P['Wv'] + P['bv']
    idx = _knn_idx(p, p, ns)
    pr = p[idx] - p[:, None, :]
    xkg = xk[idx]
    xvg = xv[idx]
    pe = pr @ P['Wp1'] + P['bp1']
    pe = jax.nn.relu(_bn(pe, P['gp'], P['bp']))
    pe = pe @ P['Wp2'] + P['bp2']
    w = xkg - xq[:, None, :] + pe
    w = jax.nn.relu(_bn(w, P['gw1'], P['bw1']))
    w = w @ P['Ww1'] + P['cw1']
    w = jax.nn.relu(_bn(w, P['gw2'], P['bw2']))
    w = w @ P['Ww2'] + P['cw2']
    w = jax.nn.softmax(w, axis=1)
    n, nsp, c = xvg.shape
    return ((xvg + pe).reshape(n, nsp, s, c // s) * w[:, :, None, :]).sum(1).reshape(n, c)


def _block(p, x, P, ns, s):
    h = jax.nn.relu(_bn(x @ P['W1'], P['g1'], P['b1']))
    h = jax.nn.relu(_bn(_pt_layer(p, h, P['tr'], ns, s), P['g2'], P['b2']))
    h = _bn(h @ P['W3'], P['g3'], P['b3'])
    return jax.nn.relu(h + x)


def _td_down(p, x, P, ns, idx):
    npnt = p[idx]
    nn = _knn_idx(npnt, p, ns)
    g = jnp.concatenate([p[nn] - npnt[:, None, :], x[nn]], axis=-1)
    h = jax.nn.relu(_bn(g @ P['W'], P['g'], P['b']))
    return npnt, jnp.max(h, axis=1)


def _point_embed(p):
    e = (2.0 ** np.arange(8)).astype(np.float32) * np.pi
    basis = np.zeros((3, 24), np.float32)
    basis[0, :8] = e
    basis[1, 8:16] = e
    basis[2, 16:] = e
    proj = p @ jnp.asarray(basis)
    return jnp.concatenate([jnp.sin(proj), jnp.cos(proj)], axis=-1)


def _relu_pallas(x):
    def body(x_ref, o_ref):
        o_ref[...] = jnp.maximum(x_ref[...], 0.0)
    return pl.pallas_call(
        body, out_shape=jax.ShapeDtypeStruct(x.shape, x.dtype))(x)


def kernel(x, params):
    p0 = x[0]
    idx1 = _fps(p0, p0.shape[0] // 4)
    idx2 = _fps(p0[idx1], p0.shape[0] // 16)
    x0 = jnp.concatenate([p0, _point_embed(p0)], axis=-1)
    h = jax.nn.relu(_bn(x0 @ params['td0']['W'], params['td0']['g'], params['td0']['b']))
    x1 = _block(p0, h, params['b0_0'], 8, 8)
    p2, h = _td_down(p0, x1, params['td1'], 16, idx1)
    h = _block(p2, h, params['b1_0'], 16, 8)
    x2 = _block(p2, h, params['b1_1'], 16, 8)
    p3, h = _td_down(p2, x2, params['td2'], 16, idx2)
    h = _block(p3, h, params['b2_0'], 16, 8)
    h = _block(p3, h, params['b2_1'], 16, 8)
    x3 = _relu_pallas(_block(p3, h, params['b2_2'], 16, 8))
    return (p0, x1, p2, x2, p3, x3)


# speedup vs baseline: 5.4306x; 2.2093x over previous
import functools

import jax
import jax.numpy as jnp
import numpy as np
from jax import lax
from jax.experimental import pallas as pl

EPS = 1e-05


def _fps_kernel_body(m, n8, p_ref, out_ref):
    px = p_ref[0]
    py = p_ref[1]
    pz = p_ref[2]
    m8 = m // 8
    fi = (lax.broadcasted_iota(jnp.int32, (8, n8), 0) * n8
          + lax.broadcasted_iota(jnp.int32, (8, n8), 1))
    fm = (lax.broadcasted_iota(jnp.int32, (8, m8), 0) * m8
          + lax.broadcasted_iota(jnp.int32, (8, m8), 1))

    def step(i, state):
        dist, qx, qy, qz, idxs = state
        dx = px - qx
        dy = py - qy
        dz = pz - qz
        d = (dx * dx + dy * dy) + dz * dz
        dist = jnp.minimum(dist, d)
        mx = jnp.max(dist)
        idx = jnp.min(jnp.where(dist == mx, fi, jnp.int32(8 * n8)))
        sel = (fi == idx).astype(jnp.float32)
        qx = jnp.sum(px * sel)
        qy = jnp.sum(py * sel)
        qz = jnp.sum(pz * sel)
        idxs = jnp.where(fm == i, idx, idxs)
        return (dist, qx, qy, qz, idxs)

    dist0 = jnp.full((8, n8), jnp.inf, jnp.float32)
    idxs0 = jnp.zeros((8, m8), jnp.int32)
    state = (dist0, px[0, 0], py[0, 0], pz[0, 0], idxs0)
    state = lax.fori_loop(1, m, step, state)
    out_ref[...] = state[4]


def _fps(p, m):
    n = p.shape[0]
    pxyz = p.T.reshape(3, 8, n // 8)
    out = pl.pallas_call(
        functools.partial(_fps_kernel_body, m, n // 8),
        out_shape=jax.ShapeDtypeStruct((8, m // 8), jnp.int32),
    )(pxyz)
    return out.reshape(m)


def _bn(x, g, b):
    return g * x / jnp.sqrt(1.0 + EPS) + b


def _knn_kernel_body(k, n, q_ref, pt_ref, out_ref):
    qb = q_ref[...]
    pt = pt_ref[...]
    QB = qb.shape[0]
    qq = jnp.sum(qb * qb, axis=1)
    pp = (pt[0] * pt[0] + pt[1] * pt[1]) + pt[2] * pt[2]
    d = (qq[:, None] + pp[None, :]
         - 2.0 * lax.dot_general(
             qb, pt, (((1,), (0,)), ((), ())),
             preferred_element_type=jnp.float32))
    li = lax.broadcasted_iota(jnp.int32, (QB, n), 1)
    vprev = jnp.full((QB, 1), -jnp.inf, jnp.float32)
    iprev = jnp.full((QB, 1), -1, jnp.int32)
    cols = []
    for _ in range(k):
        elig = (d > vprev) | ((d == vprev) & (li > iprev))
        dm = jnp.where(elig, d, jnp.inf)
        m = jnp.min(dm, axis=1, keepdims=True)
        idx = jnp.min(jnp.where(dm == m, li, jnp.int32(n)), axis=1, keepdims=True)
        cols.append(idx)
        vprev, iprev = m, idx
    out_ref[...] = jnp.concatenate(cols, axis=1)


def _knn_idx(q, p, k, QB=256):
    nq, n = q.shape[0], p.shape[0]
    QB = min(QB, nq)
    return pl.pallas_call(
        functools.partial(_knn_kernel_body, k, n),
        grid=(nq // QB,),
        in_specs=[
            pl.BlockSpec((QB, 3), lambda i: (i, 0)),
            pl.BlockSpec((3, n), lambda i: (0, 0)),
        ],
        out_specs=pl.BlockSpec((QB, k), lambda i: (i, 0)),
        out_shape=jax.ShapeDtypeStruct((nq, k), jnp.int32),
    )(q, p.T)


def _pt_layer(p, x, P, ns, s, idx):
    xq = x @ P['Wq'] + P['bq']
    xk = x @ P['Wk'] + P['bk']
    xv = x @ P['Wv'] + P['bv']
    pr = p[idx] - p[:, None, :]
    xkg = xk[idx]
    xvg = xv[idx]
    pe = pr @ P['Wp1'] + P['bp1']
    pe = jax.nn.relu(_bn(pe, P['gp'], P['bp']))
    pe = pe @ P['Wp2'] + P['bp2']
    w = xkg - xq[:, None, :] + pe
    w = jax.nn.relu(_bn(w, P['gw1'], P['bw1']))
    w = w @ P['Ww1'] + P['cw1']
    w = jax.nn.relu(_bn(w, P['gw2'], P['bw2']))
    w = w @ P['Ww2'] + P['cw2']
    w = jax.nn.softmax(w, axis=1)
    n, nsp, c = xvg.shape
    return ((xvg + pe).reshape(n, nsp, s, c // s) * w[:, :, None, :]).sum(1).reshape(n, c)


def _block(p, x, P, ns, s, idx):
    h = jax.nn.relu(_bn(x @ P['W1'], P['g1'], P['b1']))
    h = jax.nn.relu(_bn(_pt_layer(p, h, P['tr'], ns, s, idx), P['g2'], P['b2']))
    h = _bn(h @ P['W3'], P['g3'], P['b3'])
    return jax.nn.relu(h + x)


def _td_down(p, x, P, ns, idx, nn):
    npnt = p[idx]
    g = jnp.concatenate([p[nn] - npnt[:, None, :], x[nn]], axis=-1)
    h = jax.nn.relu(_bn(g @ P['W'], P['g'], P['b']))
    return npnt, jnp.max(h, axis=1)


def _point_embed(p):
    e = (2.0 ** np.arange(8)).astype(np.float32) * np.pi
    basis = np.zeros((3, 24), np.float32)
    basis[0, :8] = e
    basis[1, 8:16] = e
    basis[2, 16:] = e
    proj = p @ jnp.asarray(basis)
    return jnp.concatenate([jnp.sin(proj), jnp.cos(proj)], axis=-1)


def _relu_pallas(x):
    def body(x_ref, o_ref):
        o_ref[...] = jnp.maximum(x_ref[...], 0.0)
    return pl.pallas_call(
        body, out_shape=jax.ShapeDtypeStruct(x.shape, x.dtype))(x)


def kernel(x, params):
    p0 = x[0]
    idx1 = _fps(p0, p0.shape[0] // 4)
    idx2 = _fps(p0[idx1], p0.shape[0] // 16)
    x0 = jnp.concatenate([p0, _point_embed(p0)], axis=-1)
    h = jax.nn.relu(_bn(x0 @ params['td0']['W'], params['td0']['g'], params['td0']['b']))
    p2 = p0[idx1]
    knn0 = _knn_idx(p0, p0, 8)
    x1 = _block(p0, h, params['b0_0'], 8, 8, knn0)
    nn1 = _knn_idx(p2, p0, 16)
    p2, h = _td_down(p0, x1, params['td1'], 16, idx1, nn1)
    knn1 = _knn_idx(p2, p2, 16)
    h = _block(p2, h, params['b1_0'], 16, 8, knn1)
    x2 = _block(p2, h, params['b1_1'], 16, 8, knn1)
    p3 = p2[idx2]
    nn2 = _knn_idx(p3, p2, 16)
    p3, h = _td_down(p2, x2, params['td2'], 16, idx2, nn2)
    knn2 = _knn_idx(p3, p3, 16)
    h = _block(p3, h, params['b2_0'], 16, 8, knn2)
    h = _block(p3, h, params['b2_1'], 16, 8, knn2)
    x3 = _relu_pallas(_block(p3, h, params['b2_2'], 16, 8, knn2))
    return (p0, x1, p2, x2, p3, x3)
